# Initial kernel scaffold; baseline (speedup 1.0000x reference)
import jax
import jax.numpy as jnp
from jax.experimental import pallas as pl


def kernel(z, edge_index, batch, x, edge_weight, node_id, z_table, Wl0, Wr0, b0, Wl1, Wr1, b1, Wl2, Wr2, b2, W1, b1l, W2, b2l):
    raise NotImplementedError("write your pallas kernel here")


# speedup vs baseline: 4.9682x; 4.9682x over previous
import functools

import jax
import jax.numpy as jnp
from jax import lax
from jax.experimental import pallas as pl
from jax.experimental.pallas import tpu as pltpu
from jax.experimental.pallas import tpu_sc as plsc

N = 10000
E = 320000
H = 128
MAXZ = 1000
NGRAPH = 512

CH = 128
NCHUNK = E // CH
GCH = 80
NGC = N // GCH


def _sc_mesh():
    return plsc.VectorSubcoreMesh(core_axis_name="c", subcore_axis_name="s")


def _rows_loop(ns, sid, fn):
    def body(i, _):
        j = sid + i * ns

        @pl.when(j < NGC)
        def _():
            fn(pl.multiple_of(j * GCH, GCH))
        return 0
    lax.fori_loop(0, (NGC + ns - 1) // ns, body, 0)


@functools.lru_cache(maxsize=None)
def _make_embed_cnt():
    mesh = _sc_mesh()
    nc, ns = mesh.num_cores, mesh.num_subcores
    nw = nc * ns

    @functools.partial(
        pl.kernel,
        out_type=[
            jax.ShapeDtypeStruct((N, H), jnp.float32),
            jax.ShapeDtypeStruct((nc, N, H), jnp.float32),
        ],
        mesh=mesh,
        scratch_types=[
            pltpu.VMEM((GCH,), jnp.int32),
            pltpu.VMEM((GCH, H), jnp.float32),
            pltpu.VMEM((CH,), jnp.int32),
            pltpu.VMEM((CH, H), jnp.float32),
            pltpu.VMEM_SHARED((N, H), jnp.float32),
            pltpu.SemaphoreType.DMA,
        ],
    )
    def k(z_hbm, ztab_hbm, dst_hbm, zeros_hbm, ones_hbm, h0_hbm, cntp_hbm,
          zidx_v, grows_v, didx_v, ones_v, acc16, sem):
        cid = lax.axis_index("c")
        sid = lax.axis_index("s")
        wid = sid * nc + cid

        pltpu.sync_copy(ones_hbm, ones_v)

        _rows_loop(ns, sid, lambda base: pltpu.sync_copy(
            zeros_hbm.at[pl.ds(base, GCH)], acc16.at[pl.ds(base, GCH)]))
        plsc.subcore_barrier()

        def gbody(i, _):
            kk = wid + i * nw

            @pl.when(kk < NGC)
            def _():
                base = pl.multiple_of(kk * GCH, GCH)
                pltpu.sync_copy(z_hbm.at[pl.ds(base, GCH)], zidx_v)
                pltpu.async_copy(ztab_hbm.at[zidx_v], grows_v, sem).wait()
                pltpu.sync_copy(grows_v, h0_hbm.at[pl.ds(base, GCH)])
            return 0
        lax.fori_loop(0, (NGC + nw - 1) // nw, gbody, 0)

        def cbody(i, _):
            kk = wid + i * nw

            @pl.when(kk < NCHUNK)
            def _():
                base = pl.multiple_of(kk * CH, CH)
                pltpu.sync_copy(dst_hbm.at[pl.ds(base, CH)], didx_v)
                pltpu.sync_copy(ones_v, acc16.at[didx_v], add=True)
            return 0
        lax.fori_loop(0, (NCHUNK + nw - 1) // nw, cbody, 0)

        plsc.subcore_barrier()
        _rows_loop(ns, sid, lambda base: pltpu.sync_copy(
            acc16.at[pl.ds(base, GCH)], cntp_hbm.at[cid, pl.ds(base, GCH)]))

    return k


@functools.lru_cache(maxsize=None)
def _make_agg():
    mesh = _sc_mesh()
    nc, ns = mesh.num_cores, mesh.num_subcores
    nw = nc * ns

    @functools.partial(
        pl.kernel,
        out_type=jax.ShapeDtypeStruct((nc, N, H), jnp.float32),
        mesh=mesh,
        scratch_types=[
            pltpu.VMEM((CH,), jnp.int32),
            pltpu.VMEM((CH,), jnp.int32),
            pltpu.VMEM((CH, H), jnp.float32),
            pltpu.VMEM_SHARED((N, H), jnp.float32),
            pltpu.SemaphoreType.DMA,
        ],
    )
    def k(h_hbm, src_hbm, dst_hbm, zeros_hbm, part_hbm,
          sidx_v, didx_v, rows_v, acc, sem):
        cid = lax.axis_index("c")
        sid = lax.axis_index("s")
        wid = sid * nc + cid

        _rows_loop(ns, sid, lambda base: pltpu.sync_copy(
            zeros_hbm.at[pl.ds(base, GCH)], acc.at[pl.ds(base, GCH)]))
        plsc.subcore_barrier()

        def body(i, _):
            kk = wid + i * nw

            @pl.when(kk < NCHUNK)
            def _():
                base = pl.multiple_of(kk * CH, CH)
                pltpu.sync_copy(src_hbm.at[pl.ds(base, CH)], sidx_v)
                pltpu.sync_copy(dst_hbm.at[pl.ds(base, CH)], didx_v)
                pltpu.async_copy(h_hbm.at[sidx_v], rows_v, sem).wait()
                pltpu.sync_copy(rows_v, acc.at[didx_v], add=True)
            return 0
        lax.fori_loop(0, (NCHUNK + nw - 1) // nw, body, 0)

        plsc.subcore_barrier()
        _rows_loop(ns, sid, lambda base: pltpu.sync_copy(
            acc.at[pl.ds(base, GCH)], part_hbm.at[cid, pl.ds(base, GCH)]))

    return k


def _dot_t(a, w):
    return lax.dot_general(a, w, (((1,), (1,)), ((), ())),
                           preferred_element_type=jnp.float32)


def _tc_layer0(part, cntp, h, wl, wr, b):
    def body(part_ref, cntp_ref, h_ref, wl_ref, wr_ref, b_ref,
             h1_ref, inv_ref):
        cnt = cntp_ref[0][:, 0:1] + cntp_ref[1][:, 0:1]
        inv = 1.0 / jnp.maximum(cnt, 1.0)
        inv_ref[...] = inv
        mean = (part_ref[0] + part_ref[1]) * inv
        out = _dot_t(mean, wl_ref[...]) + _dot_t(h_ref[...], wr_ref[...])
        out = out + b_ref[...][None, :]
        h1_ref[...] = jnp.maximum(out, 0.0)

    return pl.pallas_call(
        body,
        out_shape=[
            jax.ShapeDtypeStruct((N, H), jnp.float32),
            jax.ShapeDtypeStruct((N, 1), jnp.float32),
        ],
    )(part, cntp, h, wl, wr, b)


def _tc_layer(part, inv, h, wl, wr, b):
    def body(part_ref, inv_ref, h_ref, wl_ref, wr_ref, b_ref, h1_ref):
        mean = (part_ref[0] + part_ref[1]) * inv_ref[...]
        out = _dot_t(mean, wl_ref[...]) + _dot_t(h_ref[...], wr_ref[...])
        out = out + b_ref[...][None, :]
        h1_ref[...] = jnp.maximum(out, 0.0)

    return pl.pallas_call(
        body,
        out_shape=jax.ShapeDtypeStruct((N, H), jnp.float32),
    )(part, inv, h, wl, wr, b)


def _tc_final(part, inv, h, batch, wl, wr, b, w1, b1l, w2, b2l):
    nchunks = 10
    rows_per = N // nchunks

    def body(part_ref, inv_ref, h_ref, batch_ref, wl_ref, wr_ref, b_ref,
             w1_ref, b1l_ref, w2_ref, out_ref):
        mean = (part_ref[0] + part_ref[1]) * inv_ref[...]
        h3 = _dot_t(mean, wl_ref[...]) + _dot_t(h_ref[...], wr_ref[...])
        h3 = h3 + b_ref[...][None, :]

        gi = lax.broadcasted_iota(jnp.int32, (1, NGRAPH), 1)
        ci = jnp.zeros((1, NGRAPH), jnp.int32)
        for t in range(nchunks):
            bc = batch_ref[pl.ds(t * rows_per, rows_per)]
            ci = ci + jnp.sum((bc[:, None] < gi).astype(jnp.int32),
                              axis=0, keepdims=True)
        ci_a = jnp.minimum(ci, N - 1).reshape(NGRAPH, 1)
        ci_b = jnp.minimum(ci + 1, N - 1).reshape(NGRAPH, 1)

        pa = jnp.zeros((NGRAPH, H), jnp.float32)
        pb = jnp.zeros((NGRAPH, H), jnp.float32)
        for t in range(nchunks):
            rows = h3[t * rows_per:(t + 1) * rows_per]
            nid = (lax.broadcasted_iota(jnp.int32, (NGRAPH, rows_per), 1)
                   + t * rows_per)
            oh_a = (ci_a == nid).astype(jnp.float32)
            oh_b = (ci_b == nid).astype(jnp.float32)
            pa = pa + jnp.dot(oh_a, rows, preferred_element_type=jnp.float32)
            pb = pb + jnp.dot(oh_b, rows, preferred_element_type=jnp.float32)

        p = pa * pb
        q = jnp.maximum(_dot_t(p, w1_ref[...]) + b1l_ref[...][None, :], 0.0)
        out_ref[...] = _dot_t(q, w2_ref[...])

    out = pl.pallas_call(
        body,
        out_shape=jax.ShapeDtypeStruct((NGRAPH, 1), jnp.float32),
    )(part, inv, h, batch, wl, wr, b, w1, b1l, w2)
    return out + b2l[None, :]


def kernel(z, edge_index, batch, x, edge_weight, node_id, z_table,
           Wl0, Wr0, b0, Wl1, Wr1, b1, Wl2, Wr2, b2, W1, b1l, W2, b2l):
    src = edge_index[0]
    dst = edge_index[1]
    z = z.astype(jnp.int32)
    src = src.astype(jnp.int32)
    dst = dst.astype(jnp.int32)
    batch = batch.astype(jnp.int32)
    zeros_nh = jnp.zeros((N, H), jnp.float32)
    ones_ch = jnp.ones((CH, H), jnp.float32)

    h0, cntp = _make_embed_cnt()(z, z_table, dst, zeros_nh, ones_ch)
    part0 = _make_agg()(h0, src, dst, zeros_nh)
    h1, inv = _tc_layer0(part0, cntp, h0, Wl0, Wr0, b0)
    part1 = _make_agg()(h1, src, dst, zeros_nh)
    h2 = _tc_layer(part1, inv, h1, Wl1, Wr1, b1)
    part2 = _make_agg()(h2, src, dst, zeros_nh)
    return _tc_final(part2, inv, h2, batch, Wl2, Wr2, b2, W1, b1l, W2, b2l)
